# baseline (speedup 1.0000x reference)
import functools

import jax
import jax.numpy as jnp
from jax import lax
from jax.experimental import pallas as pl
from jax.experimental.pallas import tpu as pltpu
from jax.experimental.pallas import tpu_sc as plsc

NC = 2
NS = 16
NW = NC * NS
LANES = 16
K = 128


def _sc_mesh():
    return plsc.VectorSubcoreMesh(
        core_axis_name="c", subcore_axis_name="s", num_cores=NC, num_subcores=NS
    )


def _gather_diff(node_feat, src_p, dst_p, e_pad, ch_w):
    N, D = node_feat.shape

    @functools.partial(
        pl.kernel,
        out_type=jax.ShapeDtypeStruct((e_pad, D), jnp.float32),
        mesh=_sc_mesh(),
        scratch_types=[
            pltpu.VMEM((K,), jnp.int32),
            pltpu.VMEM((K,), jnp.int32),
            pltpu.VMEM((K,), jnp.int32),
            pltpu.VMEM((K,), jnp.int32),
            pltpu.VMEM((K, D), jnp.float32),
            pltpu.VMEM((K, D), jnp.float32),
            pltpu.VMEM((K, D), jnp.float32),
            pltpu.VMEM((K, D), jnp.float32),
            pltpu.VMEM((K, D), jnp.float32),
            pltpu.VMEM((K, D), jnp.float32),
            pltpu.SemaphoreType.DMA,
            pltpu.SemaphoreType.DMA,
            pltpu.SemaphoreType.DMA,
            pltpu.SemaphoreType.DMA,
        ],
    )
    def gather_k(node_hbm, src_hbm, dst_hbm, out_hbm,
                 idxs0, idxd0, idxs1, idxd1,
                 rs0, rd0, df0, rs1, rd1, df1, gs0, gs1, ws0, ws1):
        wid = lax.axis_index("s") * NC + lax.axis_index("c")
        w0 = wid * (ch_w * K)
        idxs = (idxs0, idxs1)
        idxd = (idxd0, idxd1)
        rs = (rs0, rs1)
        rd = (rd0, rd1)
        df = (df0, df1)
        gsem = (gs0, gs1)
        wsem = (ws0, ws1)

        @pl.loop(0, ch_w)
        def _chunk(i):
            base = w0 + i * K
            pltpu.sync_copy(src_hbm.at[pl.ds(base, K)], idxs0)
            pltpu.sync_copy(dst_hbm.at[pl.ds(base, K)], idxd0)
            cs = pltpu.async_copy(node_hbm.at[idxs0], rs0, gs0)
            cd = pltpu.async_copy(node_hbm.at[idxd0], rd0, gs1)
            cs.wait()
            cd.wait()

            @pl.loop(0, K)
            def _row(r):
                for j in range(D // LANES):
                    sl = pl.ds(j * LANES, LANES)
                    rs0[r, sl] = rs0[r, sl] - rd0[r, sl]

            pltpu.sync_copy(rs0, out_hbm.at[pl.ds(base, K)])

    return gather_k(node_feat, src_p, dst_p)


def _scatter_add(msg, dst_t, n_acc, ch_w):
    E, D = msg.shape
    zeros = jnp.zeros((n_acc, D), jnp.float32)

    @functools.partial(
        pl.kernel,
        out_type=jax.ShapeDtypeStruct((NC, n_acc, D), jnp.float32),
        mesh=_sc_mesh(),
        scratch_types=[
            pltpu.VMEM((K,), jnp.int32),
            pltpu.VMEM((K,), jnp.int32),
            pltpu.VMEM((K, D), jnp.float32),
            pltpu.VMEM((K, D), jnp.float32),
            pltpu.VMEM_SHARED((n_acc, D), jnp.float32),
            pltpu.SemaphoreType.DMA,
            pltpu.SemaphoreType.DMA,
        ],
    )
    def scatter_k(msg_hbm, dst_hbm, zeros_hbm, out_hbm,
                  idx0, idx1, mv0, mv1, acc_sh, ms0, ms1):
        c = lax.axis_index("c")
        s = lax.axis_index("s")
        wid = s * NC + c
        w0 = wid * (ch_w * K)
        idx = (idx0, idx1)
        mv = (mv0, mv1)
        msem = (ms0, ms1)

        @pl.when(s == 0)
        def _():
            pltpu.sync_copy(zeros_hbm, acc_sh)

        plsc.subcore_barrier()

        @pl.loop(0, ch_w)
        def _chunk(i):
            base = w0 + i * K
            pltpu.sync_copy(dst_hbm.at[pl.ds(base, K)], idx0)
            pltpu.sync_copy(msg_hbm.at[pl.ds(base, K)], mv0)
            pltpu.sync_copy(mv0, acc_sh.at[idx0], add=True)

        plsc.subcore_barrier()

        @pl.when(s == 0)
        def _():
            pltpu.sync_copy(acc_sh, out_hbm.at[c])

    return scatter_k(msg, dst_t, zeros)


def _edge_mlp(diff, ef, w1a, w1b, b1, w2, b2, aw1a, aw1b, ab1, aw2, ab2):
    E, D = diff.shape
    DE = ef.shape[1]
    MSG = w2.shape[1]
    B = 2560
    grid = E // B

    def body(diff_ref, ef_ref, w1a_ref, w1b_ref, b1_ref, w2_ref, b2_ref,
             aw1a_ref, aw1b_ref, ab1_ref, aw2_ref, ab2_ref, out_ref):
        x = diff_ref[...]
        f = ef_ref[...]
        t1 = jnp.dot(x, w1a_ref[...], preferred_element_type=jnp.float32)
        t1 = t1 + jnp.dot(f, w1b_ref[...], preferred_element_type=jnp.float32)
        h1 = jnp.maximum(t1 + b1_ref[...], 0.0)
        msg = jnp.dot(h1, w2_ref[...], preferred_element_type=jnp.float32) + b2_ref[...]
        a1 = jnp.dot(x, aw1a_ref[...], preferred_element_type=jnp.float32)
        a1 = a1 + jnp.dot(f, aw1b_ref[...], preferred_element_type=jnp.float32)
        a1 = jnp.maximum(a1 + ab1_ref[...], 0.0)
        att = jax.nn.sigmoid(
            jnp.dot(a1, aw2_ref[...], preferred_element_type=jnp.float32) + ab2_ref[...])
        out_ref[...] = msg * att

    full = lambda shape: pl.BlockSpec(shape, lambda i: (0, 0))
    return pl.pallas_call(
        body,
        grid=(grid,),
        in_specs=[
            pl.BlockSpec((B, D), lambda i: (i, 0)),
            pl.BlockSpec((B, DE), lambda i: (i, 0)),
            full((D, MSG)), full((DE, MSG)), full((1, MSG)),
            full((MSG, MSG)), full((1, MSG)),
            full((D, MSG)), full((DE, MSG)), full((1, MSG)),
            full((MSG, MSG)), full((1, MSG)),
        ],
        out_specs=pl.BlockSpec((B, MSG), lambda i: (i, 0)),
        out_shape=jax.ShapeDtypeStruct((E, MSG), jnp.float32),
    )(diff, ef, w1a, w1b, b1, w2, b2, aw1a, aw1b, ab1, aw2, ab2)


def _gru(parts, h, wih, whh, bih, bhh):
    N, D = h.shape
    G = wih.shape[1]
    R = 2000
    grid = N // R

    def body(p_ref, h_ref, wih_ref, whh_ref, bih_ref, bhh_ref, out_ref):
        sm = p_ref[0] + p_ref[1]
        hh = h_ref[...]
        gi = jnp.dot(sm, wih_ref[...], preferred_element_type=jnp.float32) + bih_ref[...]
        gh = jnp.dot(hh, whh_ref[...], preferred_element_type=jnp.float32) + bhh_ref[...]
        i_r, i_z, i_n = gi[:, :D], gi[:, D:2 * D], gi[:, 2 * D:]
        h_r, h_z, h_n = gh[:, :D], gh[:, D:2 * D], gh[:, 2 * D:]
        r = jax.nn.sigmoid(i_r + h_r)
        z = jax.nn.sigmoid(i_z + h_z)
        n = jnp.tanh(i_n + r * h_n)
        out_ref[...] = (1.0 - z) * n + z * hh

    return pl.pallas_call(
        body,
        grid=(grid,),
        in_specs=[
            pl.BlockSpec((2, R, D), lambda i: (0, i, 0)),
            pl.BlockSpec((R, D), lambda i: (i, 0)),
            pl.BlockSpec((D, G), lambda i: (0, 0)),
            pl.BlockSpec((D, G), lambda i: (0, 0)),
            pl.BlockSpec((1, G), lambda i: (0, 0)),
            pl.BlockSpec((1, G), lambda i: (0, 0)),
        ],
        out_specs=pl.BlockSpec((R, D), lambda i: (i, 0)),
        out_shape=jax.ShapeDtypeStruct((N, D), jnp.float32),
    )(parts, h, wih, whh, bih, bhh)


def kernel(node_feat, edge_index, edge_feat, msg_w1, msg_b1, msg_w2, msg_b2,
           att_w1, att_b1, att_w2, att_b2, gru_wih, gru_whh, gru_bih, gru_bhh):
    N, D = node_feat.shape
    E = edge_index.shape[1]
    DE = edge_feat.shape[1]

    ch_w = -(-E // (NW * K))
    e_pad = NW * ch_w * K
    n_acc = 10240

    pad = e_pad - E
    src_p = jnp.concatenate([edge_index[0], jnp.zeros((pad,), jnp.int32)])
    dst_p = jnp.concatenate([edge_index[1], jnp.zeros((pad,), jnp.int32)])
    dst_trash = jnp.concatenate(
        [edge_index[1], jnp.full((pad,), N, jnp.int32)])
    ef_p = jnp.concatenate([edge_feat, jnp.zeros((pad, DE), jnp.float32)])

    diff = _gather_diff(node_feat, src_p, dst_p, e_pad, ch_w)

    msg = _edge_mlp(
        diff, ef_p,
        msg_w1[:, :D].T, msg_w1[:, D:].T, msg_b1[None, :],
        msg_w2.T, msg_b2[None, :],
        att_w1[:, :D].T, att_w1[:, D:].T, att_b1[None, :],
        att_w2.T, att_b2[None, :],
    )

    parts = _scatter_add(msg, dst_trash, n_acc, ch_w)

    return _gru(parts, node_feat, gru_wih.T, gru_whh.T,
                gru_bih[None, :], gru_bhh[None, :])


# speedup vs baseline: 1.1308x; 1.1308x over previous
import functools

import jax
import jax.numpy as jnp
from jax import lax
from jax.experimental import pallas as pl
from jax.experimental.pallas import tpu as pltpu
from jax.experimental.pallas import tpu_sc as plsc

NC = 2
NS = 16
NW = NC * NS
LANES = 16
K = 128


def _sc_mesh():
    return plsc.VectorSubcoreMesh(
        core_axis_name="c", subcore_axis_name="s", num_cores=NC, num_subcores=NS
    )


def _gather_diff(node_feat, src_p, dst_p, e_pad, ch_w):
    N, D = node_feat.shape

    @functools.partial(
        pl.kernel,
        out_type=jax.ShapeDtypeStruct((e_pad, D), jnp.float32),
        mesh=_sc_mesh(),
        scratch_types=[
            pltpu.VMEM((K,), jnp.int32),
            pltpu.VMEM((K,), jnp.int32),
            pltpu.VMEM((K,), jnp.int32),
            pltpu.VMEM((K,), jnp.int32),
            pltpu.VMEM((K, D), jnp.float32),
            pltpu.VMEM((K, D), jnp.float32),
            pltpu.VMEM((K, D), jnp.float32),
            pltpu.VMEM((K, D), jnp.float32),
            pltpu.VMEM((K, D), jnp.float32),
            pltpu.VMEM((K, D), jnp.float32),
            pltpu.SemaphoreType.DMA,
            pltpu.SemaphoreType.DMA,
            pltpu.SemaphoreType.DMA,
            pltpu.SemaphoreType.DMA,
        ],
    )
    def gather_k(node_hbm, src_hbm, dst_hbm, out_hbm,
                 idxs0, idxd0, idxs1, idxd1,
                 rs0, rd0, df0, rs1, rd1, df1, gs0, gs1, ws0, ws1):
        wid = lax.axis_index("s") * NC + lax.axis_index("c")
        w0 = wid * (ch_w * K)
        idxs = (idxs0, idxs1)
        idxd = (idxd0, idxd1)
        rs = (rs0, rs1)
        rd = (rd0, rd1)
        df = (df0, df1)
        gsem = (gs0, gs1)
        wsem = (ws0, ws1)

        def start_gather(b, i):
            base = w0 + i * K
            pltpu.sync_copy(src_hbm.at[pl.ds(base, K)], idxs[b])
            pltpu.sync_copy(dst_hbm.at[pl.ds(base, K)], idxd[b])
            return (pltpu.async_copy(node_hbm.at[idxs[b]], rs[b], gsem[b]),
                    pltpu.async_copy(node_hbm.at[idxd[b]], rd[b], gsem[b]))

        gd = [start_gather(0, 0), start_gather(1, 1)]
        wd = [None, None]
        for ii in range(ch_w):
            b = ii & 1
            gd[b][0].wait()
            gd[b][1].wait()
            if wd[b] is not None:
                wd[b].wait()

            @pl.loop(0, K)
            def _row(r, _rs=rs[b], _rd=rd[b], _df=df[b]):
                for j in range(D // LANES):
                    sl = pl.ds(j * LANES, LANES)
                    _df[r, sl] = _rs[r, sl] - _rd[r, sl]

            if ii + 2 < ch_w:
                gd[b] = start_gather(b, ii + 2)
            wd[b] = pltpu.async_copy(
                df[b], out_hbm.at[pl.ds(w0 + ii * K, K)], wsem[b])
        wd[0].wait()
        wd[1].wait()

    return gather_k(node_feat, src_p, dst_p)


def _scatter_add(msg, dst_t, n_acc, ch_w):
    E, D = msg.shape
    zeros = jnp.zeros((n_acc, D), jnp.float32)

    @functools.partial(
        pl.kernel,
        out_type=jax.ShapeDtypeStruct((NC, n_acc, D), jnp.float32),
        mesh=_sc_mesh(),
        scratch_types=[
            pltpu.VMEM((K,), jnp.int32),
            pltpu.VMEM((K,), jnp.int32),
            pltpu.VMEM((K, D), jnp.float32),
            pltpu.VMEM((K, D), jnp.float32),
            pltpu.VMEM_SHARED((n_acc, D), jnp.float32),
            pltpu.SemaphoreType.DMA,
            pltpu.SemaphoreType.DMA,
        ],
    )
    def scatter_k(msg_hbm, dst_hbm, zeros_hbm, out_hbm,
                  idx0, idx1, mv0, mv1, acc_sh, ms0, ms1):
        c = lax.axis_index("c")
        s = lax.axis_index("s")
        wid = s * NC + c
        w0 = wid * (ch_w * K)
        idx = (idx0, idx1)
        mv = (mv0, mv1)
        msem = (ms0, ms1)

        @pl.when(s == 0)
        def _():
            pltpu.sync_copy(zeros_hbm, acc_sh)

        plsc.subcore_barrier()

        @pl.loop(0, ch_w)
        def _chunk(i):
            base = w0 + i * K
            pltpu.sync_copy(dst_hbm.at[pl.ds(base, K)], idx0)
            pltpu.sync_copy(msg_hbm.at[pl.ds(base, K)], mv0)
            pltpu.sync_copy(mv0, acc_sh.at[idx0], add=True)

        plsc.subcore_barrier()

        @pl.when(s == 0)
        def _():
            pltpu.sync_copy(acc_sh, out_hbm.at[c])

    return scatter_k(msg, dst_t, zeros)


def _edge_mlp(diff, ef, w1a, w1b, b1, w2, b2, aw1a, aw1b, ab1, aw2, ab2):
    E, D = diff.shape
    DE = ef.shape[1]
    MSG = w2.shape[1]
    B = 2560
    grid = E // B

    def body(diff_ref, ef_ref, w1a_ref, w1b_ref, b1_ref, w2_ref, b2_ref,
             aw1a_ref, aw1b_ref, ab1_ref, aw2_ref, ab2_ref, out_ref):
        x = diff_ref[...]
        f = ef_ref[...]
        t1 = jnp.dot(x, w1a_ref[...], preferred_element_type=jnp.float32)
        t1 = t1 + jnp.dot(f, w1b_ref[...], preferred_element_type=jnp.float32)
        h1 = jnp.maximum(t1 + b1_ref[...], 0.0)
        msg = jnp.dot(h1, w2_ref[...], preferred_element_type=jnp.float32) + b2_ref[...]
        a1 = jnp.dot(x, aw1a_ref[...], preferred_element_type=jnp.float32)
        a1 = a1 + jnp.dot(f, aw1b_ref[...], preferred_element_type=jnp.float32)
        a1 = jnp.maximum(a1 + ab1_ref[...], 0.0)
        att = jax.nn.sigmoid(
            jnp.dot(a1, aw2_ref[...], preferred_element_type=jnp.float32) + ab2_ref[...])
        out_ref[...] = msg * att

    full = lambda shape: pl.BlockSpec(shape, lambda i: (0, 0))
    return pl.pallas_call(
        body,
        grid=(grid,),
        in_specs=[
            pl.BlockSpec((B, D), lambda i: (i, 0)),
            pl.BlockSpec((B, DE), lambda i: (i, 0)),
            full((D, MSG)), full((DE, MSG)), full((1, MSG)),
            full((MSG, MSG)), full((1, MSG)),
            full((D, MSG)), full((DE, MSG)), full((1, MSG)),
            full((MSG, MSG)), full((1, MSG)),
        ],
        out_specs=pl.BlockSpec((B, MSG), lambda i: (i, 0)),
        out_shape=jax.ShapeDtypeStruct((E, MSG), jnp.float32),
    )(diff, ef, w1a, w1b, b1, w2, b2, aw1a, aw1b, ab1, aw2, ab2)


def _gru(parts, h, wih, whh, bih, bhh):
    N, D = h.shape
    G = wih.shape[1]
    R = 2000
    grid = N // R

    def body(p_ref, h_ref, wih_ref, whh_ref, bih_ref, bhh_ref, out_ref):
        sm = p_ref[0] + p_ref[1]
        hh = h_ref[...]
        gi = jnp.dot(sm, wih_ref[...], preferred_element_type=jnp.float32) + bih_ref[...]
        gh = jnp.dot(hh, whh_ref[...], preferred_element_type=jnp.float32) + bhh_ref[...]
        i_r, i_z, i_n = gi[:, :D], gi[:, D:2 * D], gi[:, 2 * D:]
        h_r, h_z, h_n = gh[:, :D], gh[:, D:2 * D], gh[:, 2 * D:]
        r = jax.nn.sigmoid(i_r + h_r)
        z = jax.nn.sigmoid(i_z + h_z)
        n = jnp.tanh(i_n + r * h_n)
        out_ref[...] = (1.0 - z) * n + z * hh

    return pl.pallas_call(
        body,
        grid=(grid,),
        in_specs=[
            pl.BlockSpec((2, R, D), lambda i: (0, i, 0)),
            pl.BlockSpec((R, D), lambda i: (i, 0)),
            pl.BlockSpec((D, G), lambda i: (0, 0)),
            pl.BlockSpec((D, G), lambda i: (0, 0)),
            pl.BlockSpec((1, G), lambda i: (0, 0)),
            pl.BlockSpec((1, G), lambda i: (0, 0)),
        ],
        out_specs=pl.BlockSpec((R, D), lambda i: (i, 0)),
        out_shape=jax.ShapeDtypeStruct((N, D), jnp.float32),
    )(parts, h, wih, whh, bih, bhh)


def kernel(node_feat, edge_index, edge_feat, msg_w1, msg_b1, msg_w2, msg_b2,
           att_w1, att_b1, att_w2, att_b2, gru_wih, gru_whh, gru_bih, gru_bhh):
    N, D = node_feat.shape
    E = edge_index.shape[1]
    DE = edge_feat.shape[1]

    ch_w = -(-E // (NW * K))
    e_pad = NW * ch_w * K
    n_acc = 10240

    pad = e_pad - E
    src_p = jnp.concatenate([edge_index[0], jnp.zeros((pad,), jnp.int32)])
    dst_p = jnp.concatenate([edge_index[1], jnp.zeros((pad,), jnp.int32)])
    dst_trash = jnp.concatenate(
        [edge_index[1], jnp.full((pad,), N, jnp.int32)])
    ef_p = jnp.concatenate([edge_feat, jnp.zeros((pad, DE), jnp.float32)])

    diff = _gather_diff(node_feat, src_p, dst_p, e_pad, ch_w)

    msg = _edge_mlp(
        diff, ef_p,
        msg_w1[:, :D].T, msg_w1[:, D:].T, msg_b1[None, :],
        msg_w2.T, msg_b2[None, :],
        att_w1[:, :D].T, att_w1[:, D:].T, att_b1[None, :],
        att_w2.T, att_b2[None, :],
    )

    parts = _scatter_add(msg, dst_trash, n_acc, ch_w)

    return _gru(parts, node_feat, gru_wih.T, gru_whh.T,
                gru_bih[None, :], gru_bhh[None, :])


# speedup vs baseline: 1.6135x; 1.4269x over previous
import functools

import jax
import jax.numpy as jnp
from jax import lax
from jax.experimental import pallas as pl
from jax.experimental.pallas import tpu as pltpu
from jax.experimental.pallas import tpu_sc as plsc

NC = 2
NS = 16
NW = NC * NS
LANES = 16
KG = 128
KS = 128


def _sc_mesh():
    return plsc.VectorSubcoreMesh(
        core_axis_name="c", subcore_axis_name="s", num_cores=NC, num_subcores=NS
    )


def _gather_diff(node_feat, src_p, dst_p, e_pad, ch_w):
    N, D = node_feat.shape

    @functools.partial(
        pl.kernel,
        out_type=jax.ShapeDtypeStruct((e_pad, D), jnp.float32),
        mesh=_sc_mesh(),
        scratch_types=[
            pltpu.VMEM((KG,), jnp.int32),
            pltpu.VMEM((KG,), jnp.int32),
            pltpu.VMEM((KG,), jnp.int32),
            pltpu.VMEM((KG,), jnp.int32),
            pltpu.VMEM((KG, D), jnp.float32),
            pltpu.VMEM((KG, D), jnp.float32),
            pltpu.VMEM((KG, D), jnp.float32),
            pltpu.VMEM((KG, D), jnp.float32),
            pltpu.VMEM((KG, D), jnp.float32),
            pltpu.VMEM((KG, D), jnp.float32),
            pltpu.SemaphoreType.DMA,
            pltpu.SemaphoreType.DMA,
            pltpu.SemaphoreType.DMA,
            pltpu.SemaphoreType.DMA,
        ],
    )
    def gather_k(node_hbm, src_hbm, dst_hbm, out_hbm,
                 idxs0, idxd0, idxs1, idxd1,
                 rs0, rd0, df0, rs1, rd1, df1, gs0, gs1, ws0, ws1):
        wid = lax.axis_index("s") * NC + lax.axis_index("c")
        w0 = wid * (ch_w * KG)
        idxs = (idxs0, idxs1)
        idxd = (idxd0, idxd1)
        rs = (rs0, rs1)
        rd = (rd0, rd1)
        df = (df0, df1)
        gsem = (gs0, gs1)
        wsem = (ws0, ws1)

        def start_gather(b, i):
            base = w0 + i * KG
            pltpu.sync_copy(src_hbm.at[pl.ds(base, KG)], idxs[b])
            pltpu.sync_copy(dst_hbm.at[pl.ds(base, KG)], idxd[b])
            return (pltpu.async_copy(node_hbm.at[idxs[b]], rs[b], gsem[b]),
                    pltpu.async_copy(node_hbm.at[idxd[b]], rd[b], gsem[b]))

        gd = [start_gather(0, 0), start_gather(1, 1)]
        wd = [None, None]
        for ii in range(ch_w):
            b = ii & 1
            gd[b][0].wait()
            gd[b][1].wait()
            if wd[b] is not None:
                wd[b].wait()

            @pl.loop(0, KG)
            def _row(r, _rs=rs[b], _rd=rd[b], _df=df[b]):
                for j in range(D // LANES):
                    sl = pl.ds(j * LANES, LANES)
                    _df[r, sl] = _rs[r, sl] - _rd[r, sl]

            if ii + 2 < ch_w:
                gd[b] = start_gather(b, ii + 2)
            wd[b] = pltpu.async_copy(
                df[b], out_hbm.at[pl.ds(w0 + ii * KG, KG)], wsem[b])
        wd[0].wait()
        wd[1].wait()

    return gather_k(node_feat, src_p, dst_p)


def _scatter_add(msg, dst_t, n_acc, ch_w):
    E, D = msg.shape
    zeros = jnp.zeros((n_acc, D), jnp.float32)

    @functools.partial(
        pl.kernel,
        out_type=jax.ShapeDtypeStruct((NC, n_acc, D), jnp.float32),
        mesh=_sc_mesh(),
        scratch_types=[
            pltpu.VMEM((KS,), jnp.int32),
            pltpu.VMEM((KS,), jnp.int32),
            pltpu.VMEM((KS, D), jnp.float32),
            pltpu.VMEM((KS, D), jnp.float32),
            pltpu.VMEM_SHARED((n_acc, D), jnp.float32),
            pltpu.SemaphoreType.DMA,
            pltpu.SemaphoreType.DMA,
        ],
    )
    def scatter_k(msg_hbm, dst_hbm, zeros_hbm, out_hbm,
                  idx0, idx1, mv0, mv1, acc_sh, ms0, ms1):
        c = lax.axis_index("c")
        s = lax.axis_index("s")
        wid = s * NC + c
        w0 = wid * (ch_w * KS)
        idx = (idx0, idx1)
        mv = (mv0, mv1)
        msem = (ms0, ms1)

        @pl.when(s == 0)
        def _():
            pltpu.sync_copy(zeros_hbm, acc_sh)

        plsc.subcore_barrier()

        @pl.loop(0, ch_w)
        def _chunk(i):
            base = w0 + i * KS
            pltpu.sync_copy(dst_hbm.at[pl.ds(base, KS)], idx0)
            pltpu.sync_copy(msg_hbm.at[pl.ds(base, KS)], mv0)
            pltpu.sync_copy(mv0, acc_sh.at[idx0], add=True)

        plsc.subcore_barrier()

        @pl.when(s == 0)
        def _():
            pltpu.sync_copy(acc_sh, out_hbm.at[c])

    return scatter_k(msg, dst_t, zeros)


def _edge_mlp(diff, ef, w1a, w1b, b1, w2, b2, aw1a, aw1b, ab1, aw2, ab2):
    E, D = diff.shape
    DE = ef.shape[1]
    MSG = w2.shape[1]
    B = 2560
    grid = E // B

    def body(diff_ref, ef_ref, w1a_ref, w1b_ref, b1_ref, w2_ref, b2_ref,
             aw1a_ref, aw1b_ref, ab1_ref, aw2_ref, ab2_ref, out_ref):
        x = diff_ref[...]
        f = ef_ref[...]
        t1 = jnp.dot(x, w1a_ref[...], preferred_element_type=jnp.float32)
        t1 = t1 + jnp.dot(f, w1b_ref[...], preferred_element_type=jnp.float32)
        h1 = jnp.maximum(t1 + b1_ref[...], 0.0)
        msg = jnp.dot(h1, w2_ref[...], preferred_element_type=jnp.float32) + b2_ref[...]
        a1 = jnp.dot(x, aw1a_ref[...], preferred_element_type=jnp.float32)
        a1 = a1 + jnp.dot(f, aw1b_ref[...], preferred_element_type=jnp.float32)
        a1 = jnp.maximum(a1 + ab1_ref[...], 0.0)
        att = jax.nn.sigmoid(
            jnp.dot(a1, aw2_ref[...], preferred_element_type=jnp.float32) + ab2_ref[...])
        out_ref[...] = msg * att

    full = lambda shape: pl.BlockSpec(shape, lambda i: (0, 0))
    return pl.pallas_call(
        body,
        grid=(grid,),
        in_specs=[
            pl.BlockSpec((B, D), lambda i: (i, 0)),
            pl.BlockSpec((B, DE), lambda i: (i, 0)),
            full((D, MSG)), full((DE, MSG)), full((1, MSG)),
            full((MSG, MSG)), full((1, MSG)),
            full((D, MSG)), full((DE, MSG)), full((1, MSG)),
            full((MSG, MSG)), full((1, MSG)),
        ],
        out_specs=pl.BlockSpec((B, MSG), lambda i: (i, 0)),
        out_shape=jax.ShapeDtypeStruct((E, MSG), jnp.float32),
    )(diff, ef, w1a, w1b, b1, w2, b2, aw1a, aw1b, ab1, aw2, ab2)


def _gru(parts, h, wih, whh, bih, bhh):
    N, D = h.shape
    G = wih.shape[1]
    R = 2000
    grid = N // R

    def body(p_ref, h_ref, wih_ref, whh_ref, bih_ref, bhh_ref, out_ref):
        sm = p_ref[0] + p_ref[1]
        hh = h_ref[...]
        gi = jnp.dot(sm, wih_ref[...], preferred_element_type=jnp.float32) + bih_ref[...]
        gh = jnp.dot(hh, whh_ref[...], preferred_element_type=jnp.float32) + bhh_ref[...]
        i_r, i_z, i_n = gi[:, :D], gi[:, D:2 * D], gi[:, 2 * D:]
        h_r, h_z, h_n = gh[:, :D], gh[:, D:2 * D], gh[:, 2 * D:]
        r = jax.nn.sigmoid(i_r + h_r)
        z = jax.nn.sigmoid(i_z + h_z)
        n = jnp.tanh(i_n + r * h_n)
        out_ref[...] = (1.0 - z) * n + z * hh

    return pl.pallas_call(
        body,
        grid=(grid,),
        in_specs=[
            pl.BlockSpec((2, R, D), lambda i: (0, i, 0)),
            pl.BlockSpec((R, D), lambda i: (i, 0)),
            pl.BlockSpec((D, G), lambda i: (0, 0)),
            pl.BlockSpec((D, G), lambda i: (0, 0)),
            pl.BlockSpec((1, G), lambda i: (0, 0)),
            pl.BlockSpec((1, G), lambda i: (0, 0)),
        ],
        out_specs=pl.BlockSpec((R, D), lambda i: (i, 0)),
        out_shape=jax.ShapeDtypeStruct((N, D), jnp.float32),
    )(parts, h, wih, whh, bih, bhh)


def kernel(node_feat, edge_index, edge_feat, msg_w1, msg_b1, msg_w2, msg_b2,
           att_w1, att_b1, att_w2, att_b2, gru_wih, gru_whh, gru_bih, gru_bhh):
    N, D = node_feat.shape
    E = edge_index.shape[1]
    DE = edge_feat.shape[1]

    import math
    lcm = math.lcm(NW * KG, NW * KS)
    e_pad = -(-E // lcm) * lcm
    ch_g = e_pad // (NW * KG)
    ch_s = e_pad // (NW * KS)
    n_acc = 10240

    pad = e_pad - E
    pad_idx = (jnp.arange(pad, dtype=jnp.int32) * 37) % N
    src_p = jnp.concatenate([edge_index[0], pad_idx])
    dst_p = jnp.concatenate([edge_index[1], pad_idx])
    dst_trash = jnp.concatenate(
        [edge_index[1], N + (jnp.arange(pad, dtype=jnp.int32) % (n_acc - N))])
    ef_p = jnp.concatenate([edge_feat, jnp.zeros((pad, DE), jnp.float32)])

    diff = _gather_diff(node_feat, src_p, dst_p, e_pad, ch_g)

    msg = _edge_mlp(
        diff, ef_p,
        msg_w1[:, :D].T, msg_w1[:, D:].T, msg_b1[None, :],
        msg_w2.T, msg_b2[None, :],
        att_w1[:, :D].T, att_w1[:, D:].T, att_b1[None, :],
        att_w2.T, att_b2[None, :],
    )

    parts = _scatter_add(msg, dst_trash, n_acc, ch_s)

    return _gru(parts, node_feat, gru_wih.T, gru_whh.T,
                gru_bih[None, :], gru_bhh[None, :])


# speedup vs baseline: 1.8333x; 1.1362x over previous
import functools

import jax
import jax.numpy as jnp
from jax import lax
from jax.experimental import pallas as pl
from jax.experimental.pallas import tpu as pltpu
from jax.experimental.pallas import tpu_sc as plsc

NC = 2
NS = 16
NW = NC * NS
LANES = 16
KG = 128
KS = 128


def _sc_mesh():
    return plsc.VectorSubcoreMesh(
        core_axis_name="c", subcore_axis_name="s", num_cores=NC, num_subcores=NS
    )


def _gather_diff(node_feat, src_p, dst_p, e_pad, ch_w):
    N, D = node_feat.shape

    @functools.partial(
        pl.kernel,
        out_type=jax.ShapeDtypeStruct((e_pad, D), jnp.float32),
        mesh=_sc_mesh(),
        scratch_types=[
            pltpu.VMEM((KG,), jnp.int32),
            pltpu.VMEM((KG,), jnp.int32),
            pltpu.VMEM((KG,), jnp.int32),
            pltpu.VMEM((KG,), jnp.int32),
            pltpu.VMEM((KG, D), jnp.float32),
            pltpu.VMEM((KG, D), jnp.float32),
            pltpu.VMEM((KG, D), jnp.float32),
            pltpu.VMEM((KG, D), jnp.float32),
            pltpu.VMEM((KG, D), jnp.float32),
            pltpu.VMEM((KG, D), jnp.float32),
            pltpu.SemaphoreType.DMA,
            pltpu.SemaphoreType.DMA,
            pltpu.SemaphoreType.DMA,
            pltpu.SemaphoreType.DMA,
        ],
    )
    def gather_k(node_hbm, src_hbm, dst_hbm, out_hbm,
                 idxs0, idxd0, idxs1, idxd1,
                 rs0, rd0, df0, rs1, rd1, df1, gs0, gs1, ws0, ws1):
        wid = lax.axis_index("s") * NC + lax.axis_index("c")
        w0 = wid * (ch_w * KG)
        idxs = (idxs0, idxs1)
        idxd = (idxd0, idxd1)
        rs = (rs0, rs1)
        rd = (rd0, rd1)
        df = (df0, df1)
        gsem = (gs0, gs1)
        wsem = (ws0, ws1)

        def start_gather(b, i):
            base = w0 + i * KG
            pltpu.sync_copy(src_hbm.at[pl.ds(base, KG)], idxs[b])
            pltpu.sync_copy(dst_hbm.at[pl.ds(base, KG)], idxd[b])
            return (pltpu.async_copy(node_hbm.at[idxs[b]], rs[b], gsem[b]),
                    pltpu.async_copy(node_hbm.at[idxd[b]], rd[b], gsem[b]))

        gd = [start_gather(0, 0), start_gather(1, 1)]
        wd = [None, None]
        for ii in range(ch_w):
            b = ii & 1
            gd[b][0].wait()
            gd[b][1].wait()
            if wd[b] is not None:
                wd[b].wait()

            @pl.loop(0, KG)
            def _row(r, _rs=rs[b], _rd=rd[b], _df=df[b]):
                for j in range(D // LANES):
                    sl = pl.ds(j * LANES, LANES)
                    _df[r, sl] = _rs[r, sl] - _rd[r, sl]

            if ii + 2 < ch_w:
                gd[b] = start_gather(b, ii + 2)
            wd[b] = pltpu.async_copy(
                df[b], out_hbm.at[pl.ds(w0 + ii * KG, KG)], wsem[b])
        wd[0].wait()
        wd[1].wait()

    return gather_k(node_feat, src_p, dst_p)


def _scatter_add(msg, dst_t, n_acc, ch_w):
    E, D = msg.shape
    zeros = jnp.zeros((n_acc, D), jnp.float32)

    @functools.partial(
        pl.kernel,
        out_type=jax.ShapeDtypeStruct((NC, n_acc, D), jnp.float32),
        mesh=_sc_mesh(),
        scratch_types=[
            pltpu.VMEM((KS,), jnp.int32),
            pltpu.VMEM((KS,), jnp.int32),
            pltpu.VMEM((KS, D), jnp.float32),
            pltpu.VMEM((KS, D), jnp.float32),
            pltpu.VMEM_SHARED((n_acc, D), jnp.float32),
            pltpu.SemaphoreType.DMA,
            pltpu.SemaphoreType.DMA,
        ],
    )
    def scatter_k(msg_hbm, dst_hbm, zeros_hbm, out_hbm,
                  idx0, idx1, mv0, mv1, acc_sh, ms0, ms1):
        c = lax.axis_index("c")
        s = lax.axis_index("s")
        wid = s * NC + c
        w0 = wid * (ch_w * KS)
        idx = (idx0, idx1)
        mv = (mv0, mv1)
        msem = (ms0, ms1)

        @pl.when(s == 0)
        def _():
            pltpu.sync_copy(zeros_hbm, acc_sh)

        plsc.subcore_barrier()

        def start_read(b, i):
            base = w0 + i * KS
            return (pltpu.async_copy(dst_hbm.at[pl.ds(base, KS)], idx[b], msem[b]),
                    pltpu.async_copy(msg_hbm.at[pl.ds(base, KS)], mv[b], msem[b]))

        rd = [start_read(0, 0), start_read(1, 1)]
        for ii in range(ch_w):
            b = ii & 1
            rd[b][0].wait()
            rd[b][1].wait()
            pltpu.sync_copy(mv[b], acc_sh.at[idx[b]], add=True)
            if ii + 2 < ch_w:
                rd[b] = start_read(b, ii + 2)

        plsc.subcore_barrier()

        @pl.when(s == 0)
        def _():
            pltpu.sync_copy(acc_sh, out_hbm.at[c])

    return scatter_k(msg, dst_t, zeros)


def _edge_mlp(diff, ef, w1a, w1b, b1, w2, b2, aw1a, aw1b, ab1, aw2, ab2):
    E, D = diff.shape
    DE = ef.shape[1]
    MSG = w2.shape[1]
    B = 2560
    grid = E // B

    def body(diff_ref, ef_ref, w1a_ref, w1b_ref, b1_ref, w2_ref, b2_ref,
             aw1a_ref, aw1b_ref, ab1_ref, aw2_ref, ab2_ref, out_ref):
        x = diff_ref[...].astype(jnp.bfloat16)
        f = ef_ref[...].astype(jnp.bfloat16)
        t1 = jnp.dot(x, w1a_ref[...], preferred_element_type=jnp.float32)
        t1 = t1 + jnp.dot(f, w1b_ref[...], preferred_element_type=jnp.float32)
        h1 = jnp.maximum(t1 + b1_ref[...], 0.0).astype(jnp.bfloat16)
        msg = jnp.dot(h1, w2_ref[...], preferred_element_type=jnp.float32) + b2_ref[...]
        a1 = jnp.dot(x, aw1a_ref[...], preferred_element_type=jnp.float32)
        a1 = a1 + jnp.dot(f, aw1b_ref[...], preferred_element_type=jnp.float32)
        a1 = jnp.maximum(a1 + ab1_ref[...], 0.0).astype(jnp.bfloat16)
        att = jax.nn.sigmoid(
            jnp.dot(a1, aw2_ref[...], preferred_element_type=jnp.float32) + ab2_ref[...])
        out_ref[...] = msg * att

    full = lambda shape: pl.BlockSpec(shape, lambda i: (0, 0))
    return pl.pallas_call(
        body,
        grid=(grid,),
        in_specs=[
            pl.BlockSpec((B, D), lambda i: (i, 0)),
            pl.BlockSpec((B, DE), lambda i: (i, 0)),
            full((D, MSG)), full((DE, MSG)), full((1, MSG)),
            full((MSG, MSG)), full((1, MSG)),
            full((D, MSG)), full((DE, MSG)), full((1, MSG)),
            full((MSG, MSG)), full((1, MSG)),
        ],
        out_specs=pl.BlockSpec((B, MSG), lambda i: (i, 0)),
        out_shape=jax.ShapeDtypeStruct((E, MSG), jnp.float32),
    )(diff, ef, w1a, w1b, b1, w2, b2, aw1a, aw1b, ab1, aw2, ab2)


def _gru(parts, h, wih, whh, bih, bhh):
    N, D = h.shape
    G = wih.shape[1]
    R = 2000
    grid = N // R

    def body(p_ref, h_ref, wih_ref, whh_ref, bih_ref, bhh_ref, out_ref):
        sm = p_ref[0] + p_ref[1]
        hh = h_ref[...]
        gi = jnp.dot(sm, wih_ref[...], preferred_element_type=jnp.float32) + bih_ref[...]
        gh = jnp.dot(hh, whh_ref[...], preferred_element_type=jnp.float32) + bhh_ref[...]
        i_r, i_z, i_n = gi[:, :D], gi[:, D:2 * D], gi[:, 2 * D:]
        h_r, h_z, h_n = gh[:, :D], gh[:, D:2 * D], gh[:, 2 * D:]
        r = jax.nn.sigmoid(i_r + h_r)
        z = jax.nn.sigmoid(i_z + h_z)
        n = jnp.tanh(i_n + r * h_n)
        out_ref[...] = (1.0 - z) * n + z * hh

    return pl.pallas_call(
        body,
        grid=(grid,),
        in_specs=[
            pl.BlockSpec((2, R, D), lambda i: (0, i, 0)),
            pl.BlockSpec((R, D), lambda i: (i, 0)),
            pl.BlockSpec((D, G), lambda i: (0, 0)),
            pl.BlockSpec((D, G), lambda i: (0, 0)),
            pl.BlockSpec((1, G), lambda i: (0, 0)),
            pl.BlockSpec((1, G), lambda i: (0, 0)),
        ],
        out_specs=pl.BlockSpec((R, D), lambda i: (i, 0)),
        out_shape=jax.ShapeDtypeStruct((N, D), jnp.float32),
    )(parts, h, wih, whh, bih, bhh)


def kernel(node_feat, edge_index, edge_feat, msg_w1, msg_b1, msg_w2, msg_b2,
           att_w1, att_b1, att_w2, att_b2, gru_wih, gru_whh, gru_bih, gru_bhh):
    N, D = node_feat.shape
    E = edge_index.shape[1]
    DE = edge_feat.shape[1]

    import math
    lcm = math.lcm(NW * KG, NW * KS)
    e_pad = -(-E // lcm) * lcm
    ch_g = e_pad // (NW * KG)
    ch_s = e_pad // (NW * KS)
    n_acc = 10240

    pad = e_pad - E
    pad_idx = (jnp.arange(pad, dtype=jnp.int32) * 37) % N
    src_p = jnp.concatenate([edge_index[0], pad_idx])
    dst_p = jnp.concatenate([edge_index[1], pad_idx])
    dst_trash = jnp.concatenate(
        [edge_index[1], N + (jnp.arange(pad, dtype=jnp.int32) % (n_acc - N))])
    ef_p = jnp.concatenate([edge_feat, jnp.zeros((pad, DE), jnp.float32)])

    diff = _gather_diff(node_feat, src_p, dst_p, e_pad, ch_g)

    bf = jnp.bfloat16
    msg = _edge_mlp(
        diff, ef_p,
        msg_w1[:, :D].T.astype(bf), msg_w1[:, D:].T.astype(bf), msg_b1[None, :],
        msg_w2.T.astype(bf), msg_b2[None, :],
        att_w1[:, :D].T.astype(bf), att_w1[:, D:].T.astype(bf), att_b1[None, :],
        att_w2.T.astype(bf), att_b2[None, :],
    )

    parts = _scatter_add(msg, dst_trash, n_acc, ch_s)

    return _gru(parts, node_feat, gru_wih.T, gru_whh.T,
                gru_bih[None, :], gru_bhh[None, :])
